# baseline (speedup 1.0000x reference)
import functools

import jax
import jax.numpy as jnp
from jax import lax
from jax.experimental import pallas as pl
from jax.experimental.pallas import tpu as pltpu
from jax.experimental.pallas import tpu_sc as plsc

NC = 2
NS = 16
NW = NC * NS
LANES = 16


@functools.lru_cache(maxsize=None)
def _build(B, L, D, V):
    assert B % (2 * NW) == 0
    s_per = B // NW
    c0 = min(L, 128) - (min(L, 128) % 8)
    chunks = []
    off = 0
    while off < L:
        n = min(c0, L - off)
        chunks.append((off, n))
        off += n

    mesh = plsc.VectorSubcoreMesh(
        core_axis_name="c", subcore_axis_name="s",
        num_cores=NC, num_subcores=NS,
    )

    @functools.partial(
        pl.kernel,
        out_type=jax.ShapeDtypeStruct((B, L, D), jnp.float32),
        mesh=mesh,
        scratch_types=[
            pltpu.VMEM((s_per, L), jnp.int32),
            pltpu.VMEM((L, D), jnp.float32),
            pltpu.VMEM((2, L, D), jnp.float32),
            pltpu.SemaphoreType.DMA,
            pltpu.SemaphoreType.DMA,
            pltpu.SemaphoreType.DMA,
        ],
    )
    def emb_kernel(x_hbm, tok_hbm, pos_hbm, out_hbm,
                   idx_v, pos_v, rows_v, sem_g, sem_o0, sem_o1):
        cid = lax.axis_index("c")
        sid = lax.axis_index("s")
        wid = sid * NC + cid
        base = wid * s_per
        sems_o = (sem_o0, sem_o1)

        pltpu.sync_copy(pos_hbm, pos_v)
        pltpu.sync_copy(x_hbm.at[pl.ds(base, s_per)], idx_v)

        def fire_gather(i, b):
            for (off, n) in chunks:
                pltpu.async_copy(
                    tok_hbm.at[idx_v.at[i, pl.ds(off, n)]],
                    rows_v.at[b, pl.ds(off, n)],
                    sem_g,
                )

        def wait_gather(i, b):
            for (off, n) in chunks:
                pltpu.make_async_copy(
                    tok_hbm.at[idx_v.at[i, pl.ds(off, n)]],
                    rows_v.at[b, pl.ds(off, n)],
                    sem_g,
                ).wait()

        def fire_out(i, b):
            pltpu.async_copy(rows_v.at[b], out_hbm.at[base + i], sems_o[b])

        def wait_out(b):
            pltpu.make_async_copy(
                rows_v.at[b], out_hbm.at[base], sems_o[b],
            ).wait()

        def add_pos(b):
            def row_body(r, c2):
                for j in range(D // LANES):
                    sl = pl.ds(j * LANES, LANES)
                    plsc.addupdate(rows_v.at[b, r, sl], pos_v[r, sl])
                return c2
            lax.fori_loop(0, L, row_body, 0, unroll=2)

        fire_gather(0, 0)

        def outer(i0, carry):
            wait_gather(i0, 0)
            pl.when(i0 > 0)(lambda: wait_out(1))
            fire_gather(i0 + 1, 1)
            add_pos(0)
            fire_out(i0, 0)
            i1 = i0 + 1
            wait_gather(i1, 1)
            wait_out(0)
            pl.when(i1 + 1 < s_per)(lambda: fire_gather(i1 + 1, 0))
            add_pos(1)
            fire_out(i1, 1)
            return carry

        lax.fori_loop(0, s_per // 2, lambda t, c: outer(t * 2, c), 0)
        wait_out(1)

    return emb_kernel


def kernel(x, token_table, pos_table):
    B, L = x.shape
    V, D = token_table.shape
    fn = _build(B, L, D, V)
    return fn(x.astype(jnp.int32), token_table, pos_table)


# speedup vs baseline: 1.2029x; 1.2029x over previous
import functools

import jax
import jax.numpy as jnp
from jax import lax
from jax.experimental import pallas as pl
from jax.experimental.pallas import tpu as pltpu
from jax.experimental.pallas import tpu_sc as plsc

NC = 2
NS = 16
NW = NC * NS
LANES = 16


@functools.lru_cache(maxsize=None)
def _build(B, L, D, V):
    assert B % (2 * NW) == 0
    s_per = B // NW
    c0 = min(L, 128) - (min(L, 128) % 8)
    chunks = []
    off = 0
    while off < L:
        n = min(c0, L - off)
        chunks.append((off, n))
        off += n

    mesh = plsc.VectorSubcoreMesh(
        core_axis_name="c", subcore_axis_name="s",
        num_cores=NC, num_subcores=NS,
    )

    @functools.partial(
        pl.kernel,
        out_type=jax.ShapeDtypeStruct((B, L, D), jnp.float32),
        mesh=mesh,
        scratch_types=[
            pltpu.VMEM((s_per // 2, L), jnp.int32),
            pltpu.VMEM((L, D), jnp.float32),
            pltpu.VMEM((3, L, D), jnp.float32),
            pltpu.SemaphoreType.DMA,
            pltpu.SemaphoreType.DMA,
            pltpu.SemaphoreType.DMA,
            pltpu.SemaphoreType.DMA,
        ],
    )
    def emb_kernel(x_hbm, tok_hbm, pos_hbm, out_hbm,
                   idx_v, pos_v, rows_v, sem_g, sem_o0, sem_o1, sem_o2):
        cid = lax.axis_index("c")
        sid = lax.axis_index("s")
        wid = sid * NC + cid
        base = wid * s_per
        sems_o = (sem_o0, sem_o1, sem_o2)

        hs = s_per // 2
        pltpu.sync_copy(pos_hbm, pos_v)
        pltpu.sync_copy(x_hbm.at[pl.ds(base, hs)], idx_v)

        def fire_gather(i, b):
            for (off, n) in chunks:
                pltpu.async_copy(
                    tok_hbm.at[idx_v.at[lax.rem(i, hs), pl.ds(off, n)]],
                    rows_v.at[b, pl.ds(off, n)],
                    sem_g,
                )

        def wait_gather(i, b):
            for (off, n) in chunks:
                pltpu.make_async_copy(
                    tok_hbm.at[idx_v.at[lax.rem(i, hs), pl.ds(off, n)]],
                    rows_v.at[b, pl.ds(off, n)],
                    sem_g,
                ).wait()

        def fire_out(i, b):
            pltpu.async_copy(rows_v.at[b], out_hbm.at[base + i], sems_o[b])

        def wait_out(b):
            pltpu.make_async_copy(
                rows_v.at[b], out_hbm.at[base], sems_o[b],
            ).wait()

        def add_pos(b):
            def row_body(r, c2):
                for j in range(D // LANES):
                    sl = pl.ds(j * LANES, LANES)
                    plsc.addupdate(rows_v.at[b, r, sl], pos_v[r, sl])
                return c2
            lax.fori_loop(0, L, row_body, 0, unroll=2)

        fire_gather(0, 0)
        fire_gather(1, 1)

        def step(i, b):
            wait_gather(i, b)
            add_pos(b)
            fire_out(i, b)
            b2 = (b + 2) % 3
            pl.when(i >= 1)(lambda: wait_out(b2))
            pl.when(i + 2 == hs)(
                lambda: pltpu.sync_copy(x_hbm.at[pl.ds(base + hs, hs)], idx_v))
            pl.when(i + 2 < s_per)(lambda: fire_gather(i + 2, b2))

        def outer(t, carry):
            i0 = t * 3
            step(i0, 0)
            step(i0 + 1, 1)
            step(i0 + 2, 2)
            return carry

        main = s_per - s_per % 3
        lax.fori_loop(0, s_per // 3, outer, 0)
        for i in range(main, s_per):
            step(i, i % 3)
        wait_out((s_per - 1) % 3)

    return emb_kernel


def kernel(x, token_table, pos_table):
    B, L = x.shape
    V, D = token_table.shape
    fn = _build(B, L, D, V)
    return fn(x.astype(jnp.int32), token_table, pos_table)
